# baseline (speedup 1.0000x reference)
import functools

import jax
import jax.numpy as jnp
from jax.experimental import pallas as pl
from jax.experimental.pallas import tpu as pltpu
from jax.experimental.pallas import tpu_sc as plsc

MAX_LEN = 512


def kernel(q, k, rel_pos_enc):
    seq_len_q = q.shape[1]
    seq_len_k = k.shape[1]
    d = rel_pos_enc.shape[1]

    info = plsc.get_sparse_core_info()
    nc, ns = info.num_cores, info.num_subcores
    nw = nc * ns
    rows_per_w = seq_len_q // nw

    mesh = plsc.VectorSubcoreMesh(core_axis_name="c", subcore_axis_name="s")

    @functools.partial(
        pl.kernel,
        mesh=mesh,
        out_type=jax.ShapeDtypeStruct((seq_len_q, seq_len_k, d), rel_pos_enc.dtype),
        compiler_params=pltpu.CompilerParams(use_tc_tiling_on_sc=False),
        scratch_types=[pltpu.SemaphoreType.DMA],
    )
    def run(table_hbm, out_hbm, sem):
        wid = jax.lax.axis_index("s") * nc + jax.lax.axis_index("c")
        base = wid * rows_per_w

        def mk(r):
            i = base + r
            start = (MAX_LEN - 1) - i
            return pltpu.make_async_copy(
                table_hbm.at[pl.ds(start, seq_len_k), :], out_hbm.at[i], sem)

        for r in range(rows_per_w):
            mk(r).start()
        for r in range(rows_per_w):
            mk(r).wait()

    return run(rel_pos_enc)


# speedup vs baseline: 21.3387x; 21.3387x over previous
import functools

import jax
import jax.numpy as jnp
from jax.experimental import pallas as pl
from jax.experimental.pallas import tpu as pltpu
from jax.experimental.pallas import tpu_sc as plsc

MAX_LEN = 512
CHUNK_J = 256


def kernel(q, k, rel_pos_enc):
    seq_len_q = q.shape[1]
    seq_len_k = k.shape[1]
    d = rel_pos_enc.shape[1]

    info = plsc.get_sparse_core_info()
    nc, ns = info.num_cores, info.num_subcores
    nw = nc * ns
    rows_per_w = seq_len_q // nw
    n_half = seq_len_k // CHUNK_J
    win_rows = CHUNK_J + rows_per_w - 1

    mesh = plsc.VectorSubcoreMesh(core_axis_name="c", subcore_axis_name="s")

    @functools.partial(
        pl.kernel,
        mesh=mesh,
        out_type=jax.ShapeDtypeStruct((seq_len_q, seq_len_k, d), rel_pos_enc.dtype),
        compiler_params=pltpu.CompilerParams(use_tc_tiling_on_sc=False),
        scratch_types=[
            pltpu.VMEM((win_rows, d), jnp.float32),
            pltpu.SemaphoreType.DMA,
        ],
    )
    def run(table_hbm, out_hbm, win, sem_out):
        wid = jax.lax.axis_index("s") * nc + jax.lax.axis_index("c")
        base = wid * rows_per_w

        for h in range(n_half):
            w0 = CHUNK_J * h + (MAX_LEN - 1) - (rows_per_w - 1) - base
            pltpu.sync_copy(table_hbm.at[pl.ds(w0, win_rows), :], win)

            def mk(r):
                return pltpu.make_async_copy(
                    win.at[pl.ds(rows_per_w - 1 - r, CHUNK_J), :],
                    out_hbm.at[base + r, pl.ds(CHUNK_J * h, CHUNK_J), :],
                    sem_out,
                )

            for r in range(rows_per_w):
                mk(r).start()
            for r in range(rows_per_w):
                mk(r).wait()

    return run(rel_pos_enc)


# speedup vs baseline: 100.7045x; 4.7193x over previous
import functools

import jax
import jax.numpy as jnp
from jax.experimental import pallas as pl
from jax.experimental.pallas import tpu as pltpu

MAX_LEN = 512
INFLIGHT = 8


def _dma_kernel(t_ref, out_ref, t8_ref, sem, *, seq_len_q, seq_len_k, max_len,
                inflight):
    tv = t_ref[...]
    for c in range(8):
        t8_ref[c] = pltpu.roll(tv, t_ref.shape[0] - c, 0) if c else tv

    def mk(i):
        s = (max_len - 1) - i
        c = jax.lax.rem(s, 8)
        aligned = pl.multiple_of(s - c, 8)
        return pltpu.make_async_copy(
            t8_ref.at[c, pl.ds(aligned, seq_len_k), :],
            out_ref.at[i],
            sem,
        )

    def body(i, carry):
        mk(i).start()

        @pl.when(i >= inflight)
        def _():
            mk(i - inflight).wait()

        return carry

    jax.lax.fori_loop(0, seq_len_q, body, 0)

    def tail(i, carry):
        mk(seq_len_q - inflight + i).wait()
        return carry

    jax.lax.fori_loop(0, inflight, tail, 0)


def kernel(q, k, rel_pos_enc):
    seq_len_q = q.shape[1]
    seq_len_k = k.shape[1]
    d = rel_pos_enc.shape[1]
    n = rel_pos_enc.shape[0]

    n_pad = ((n + 7) // 8) * 8
    padded = jnp.pad(rel_pos_enc, ((0, n_pad - n), (0, 0)))

    body = functools.partial(
        _dma_kernel,
        seq_len_q=seq_len_q,
        seq_len_k=seq_len_k,
        max_len=MAX_LEN,
        inflight=INFLIGHT,
    )
    return pl.pallas_call(
        body,
        in_specs=[
            pl.BlockSpec(memory_space=pltpu.MemorySpace.VMEM),
        ],
        out_specs=pl.BlockSpec(memory_space=pltpu.MemorySpace.HBM),
        out_shape=jax.ShapeDtypeStruct((seq_len_q, seq_len_k, d), rel_pos_enc.dtype),
        scratch_shapes=[
            pltpu.VMEM((8, n_pad, d), rel_pos_enc.dtype),
            pltpu.SemaphoreType.DMA,
        ],
    )(padded)
